# baseline (speedup 1.0000x reference)
import functools

import jax
import jax.numpy as jnp
from jax import lax
from jax.experimental import pallas as pl
from jax.experimental.pallas import tpu as pltpu
from jax.experimental.pallas import tpu_sc as plsc

L = 16
NS = 16
CHUNK = 96
HALF = 128
NROWS = 3
NIDX = 6
NPAD = 10112


def _mm_body(x_ref, w_ref, o_ref, *, relu):
    x = x_ref[...]
    if relu:
        x = jnp.maximum(x, 0.0)
    o_ref[0] = jnp.dot(x, w_ref[...], preferred_element_type=jnp.float32)


def _mm_stacked(x, w, relu, bn):
    n, fd = x.shape
    return pl.pallas_call(
        functools.partial(_mm_body, relu=relu),
        grid=(n // bn, 2),
        in_specs=[
            pl.BlockSpec((bn, fd), lambda i, j: (i, 0)),
            pl.BlockSpec((fd, HALF), lambda i, j: (0, j)),
        ],
        out_specs=pl.BlockSpec((1, bn, HALF), lambda i, j: (j, i, 0)),
        out_shape=jax.ShapeDtypeStruct((2, n, HALF), jnp.float32),
    )(x, w)


def _spmm_sc(xv, pk, pw, zrows):
    ct = pk.shape[2]
    rpt = NPAD // NS
    mesh = plsc.VectorSubcoreMesh(core_axis_name="c", subcore_axis_name="s")

    @functools.partial(
        pl.kernel,
        out_type=jax.ShapeDtypeStruct((2, NPAD, HALF), jnp.float32),
        mesh=mesh,
        scratch_types=[
            pltpu.MemorySpace.VMEM_SHARED((NPAD, HALF), jnp.float32),
            [pltpu.VMEM((2, CHUNK), jnp.int32)] * NIDX,
            [pltpu.VMEM((CHUNK,), jnp.float32)] * NIDX,
            [pltpu.VMEM((CHUNK, HALF), jnp.float32)] * NROWS,
            [pltpu.SemaphoreType.DMA] * NIDX,
            [pltpu.SemaphoreType.DMA] * NROWS,
            [pltpu.SemaphoreType.DMA] * NROWS,
        ],
    )
    def k(xv_hbm, pk_hbm, pw_hbm, z_hbm, out_hbm,
          acc, ibuf, wbuf, rows, isem, gsem, ssem):
        c = lax.axis_index("c")
        s = lax.axis_index("s")
        row0 = pl.multiple_of(s * rpt, 8)
        pltpu.sync_copy(z_hbm, acc.at[pl.ds(row0, rpt)])
        plsc.subcore_barrier()

        def prefetch(j, q):
            pltpu.async_copy(pk_hbm.at[c, s, j], ibuf[q], isem[q])
            pltpu.async_copy(pw_hbm.at[s, j], wbuf[q], isem[q])

        def wait_prefetch(j, q):
            pltpu.make_async_copy(pk_hbm.at[c, s, j], ibuf[q], isem[q]).wait()
            pltpu.make_async_copy(pw_hbm.at[s, j], wbuf[q], isem[q]).wait()

        def gather(q, r):
            pltpu.async_copy(xv_hbm.at[ibuf[q].at[0]], rows[r], gsem[r])

        def wait_gather(q, r):
            pltpu.make_async_copy(xv_hbm.at[ibuf[q].at[0]], rows[r],
                                  gsem[r]).wait()

        def scatter(q, r):
            pltpu.async_copy(rows[r], acc.at[ibuf[q].at[1]], ssem[r], add=True)

        def wait_scatter(q, r):
            pltpu.make_async_copy(rows[r], acc.at[ibuf[q].at[1]],
                                  ssem[r]).wait()

        def scale(q, r):
            def group_body(gi, gcarry):
                base = gi * L
                wvec = wbuf[q][pl.ds(base, L)]
                for i in range(L):
                    wv = jnp.full((L,), wvec[i], jnp.float32)
                    e = base + i
                    for g in range(HALF // L):
                        sl = pl.ds(g * L, L)
                        rows[r][e, sl] = rows[r][e, sl] * wv
                return gcarry

            lax.fori_loop(0, CHUNK // L, group_body, 0)

        for j in range(4):
            prefetch(j, j)
        for j in range(2):
            wait_prefetch(j, j)
            gather(j, j)

        def block_body(tt, carry):
            j0 = tt * NIDX
            for b in range(NIDX):
                j = j0 + b
                q, r = b, b % NROWS
                q2, r2 = (b + 2) % NIDX, (b + 2) % NROWS
                q4 = (b + 4) % NIDX
                wait_gather(q, r)
                scale(q, r)
                scatter(q, r)
                if b == 0:
                    @pl.when(tt >= 1)
                    def _():
                        wait_scatter((b - 1) % NIDX, (b - 1) % NROWS)
                else:
                    wait_scatter(b - 1, (b - 1) % NROWS)
                def launch_gather():
                    wait_prefetch(j + 2, q2)
                    gather(q2, r2)
                if b >= 4:
                    @pl.when(j + 2 < ct)
                    def _():
                        launch_gather()
                else:
                    launch_gather()
                def launch_pref():
                    prefetch(j + 4, q4)
                if b <= 1:
                    launch_pref()
                else:
                    @pl.when(j + 4 < ct)
                    def _():
                        launch_pref()
            return carry

        lax.fori_loop(0, ct // NIDX, block_body, 0)
        wait_scatter((ct - 1) % NIDX, (ct - 1) % NROWS)
        plsc.subcore_barrier()
        pltpu.sync_copy(acc.at[pl.ds(row0, rpt)],
                        out_hbm.at[c, pl.ds(row0, rpt)])

    return k(xv, pk, pw, zrows)


def kernel(edge_index, edge_weight, feat, W1, W2):
    n = feat.shape[0]
    e = edge_weight.shape[0]
    align = NS * CHUNK * NIDX
    e_pad = -(-e // align) * align
    ct = e_pad // (NS * CHUNK)

    dst = edge_index[0].astype(jnp.int32)
    src = edge_index[1].astype(jnp.int32)
    w = edge_weight.astype(jnp.float32)
    pad = e_pad - e
    src_p = jnp.pad(src, (0, pad))
    dst_p = jnp.pad(dst, (0, pad))
    w_p = jnp.pad(w, (0, pad))

    def per_core(sc):
        return jnp.stack(
            [sc.reshape(NS, ct, CHUNK), dst_p.reshape(NS, ct, CHUNK)], axis=2)

    pk = jnp.stack([per_core(src_p), per_core(src_p + NPAD)])
    pw = w_p.reshape(NS, ct, CHUNK)
    zrows = jnp.zeros((NPAD // NS, HALF), jnp.float32)

    def as_table(xs):
        m = xs.shape[1]
        xs = jnp.pad(xs, ((0, 0), (0, NPAD - m), (0, 0)))
        return xs.reshape(2 * NPAD, HALF)

    def unstack(ys):
        return jnp.concatenate([ys[0], ys[1]], axis=1)

    x1 = _mm_stacked(feat, W1, relu=False, bn=n // 10)
    y1 = _spmm_sc(as_table(x1), pk, pw, zrows)
    x2 = _mm_stacked(unstack(y1), W2, relu=True, bn=NPAD // 8)
    y2 = _spmm_sc(as_table(x2), pk, pw, zrows)
    return unstack(y2)[:n]


# speedup vs baseline: 1.2675x; 1.2675x over previous
import functools

import jax
import jax.numpy as jnp
from jax import lax
from jax.experimental import pallas as pl
from jax.experimental.pallas import tpu as pltpu
from jax.experimental.pallas import tpu_sc as plsc

L = 16
NS = 16
CHUNK = 128
HALF = 128


def _mm_body(x_ref, w_ref, o_ref, *, relu):
    x = x_ref[...]
    if relu:
        x = jnp.maximum(x, 0.0)
    o_ref[0] = jnp.dot(x, w_ref[...], preferred_element_type=jnp.float32)


def _mm_stacked(x, w, relu):
    n, fd = x.shape
    bn = n // 10
    return pl.pallas_call(
        functools.partial(_mm_body, relu=relu),
        grid=(n // bn, 2),
        in_specs=[
            pl.BlockSpec((bn, fd), lambda i, j: (i, 0)),
            pl.BlockSpec((fd, HALF), lambda i, j: (0, j)),
        ],
        out_specs=pl.BlockSpec((1, bn, HALF), lambda i, j: (j, i, 0)),
        out_shape=jax.ShapeDtypeStruct((2, n, HALF), jnp.float32),
    )(x, w)


def _spmm_sc(n_pad, xv, src2, dst_r, w_r, zrows):
    ct = dst_r.shape[1]
    rpt = n_pad // NS
    mesh = plsc.VectorSubcoreMesh(core_axis_name="c", subcore_axis_name="s")

    @functools.partial(
        pl.kernel,
        out_type=jax.ShapeDtypeStruct((n_pad, 2 * HALF), jnp.float32),
        mesh=mesh,
        scratch_types=[
            pltpu.MemorySpace.VMEM_SHARED((n_pad, HALF), jnp.float32),
            pltpu.VMEM((ct, CHUNK), jnp.int32),
            pltpu.VMEM((ct, CHUNK), jnp.int32),
            pltpu.VMEM((ct * CHUNK,), jnp.float32),
            pltpu.VMEM((CHUNK, HALF), jnp.float32),
            pltpu.SemaphoreType.DMA,
        ],
    )
    def k(xv_hbm, src_hbm, dst_hbm, w_hbm, z_hbm, out_hbm,
          acc, src_v, dst_v, w_v, rows_v, sem):
        c = lax.axis_index("c")
        s = lax.axis_index("s")
        row0 = pl.multiple_of(s * rpt, 8)
        pltpu.sync_copy(z_hbm, acc.at[pl.ds(row0, rpt)])
        pltpu.sync_copy(src_hbm.at[c, s], src_v)
        pltpu.sync_copy(dst_hbm.at[s], dst_v)
        pltpu.sync_copy(w_hbm.at[s], w_v)
        plsc.subcore_barrier()

        def chunk_body(j, carry):
            pltpu.async_copy(xv_hbm.at[src_v.at[j]], rows_v, sem).wait()

            def group_body(gi, gcarry):
                base = gi * L
                wvec = w_v[pl.ds(j * CHUNK + base, L)]
                for i in range(L):
                    wv = jnp.full((L,), wvec[i], jnp.float32)
                    e = base + i
                    for g in range(HALF // L):
                        sl = pl.ds(g * L, L)
                        rows_v[e, sl] = rows_v[e, sl] * wv
                return gcarry

            lax.fori_loop(0, CHUNK // L, group_body, 0)
            pltpu.sync_copy(rows_v, acc.at[dst_v.at[j]], add=True)
            return carry

        lax.fori_loop(0, ct, chunk_body, 0)
        plsc.subcore_barrier()
        pltpu.sync_copy(
            acc.at[pl.ds(row0, rpt)],
            out_hbm.at[pl.ds(row0, rpt), pl.ds(c * HALF, HALF)],
        )

    return k(xv, src2, dst_r, w_r, zrows)


def kernel(edge_index, edge_weight, feat, W1, W2):
    n = feat.shape[0]
    e = edge_weight.shape[0]
    n_pad = -(-n // 640) * 640
    per_tile = -(-e // (NS * CHUNK)) * CHUNK
    e_pad = per_tile * NS
    ct = per_tile // CHUNK

    dst = edge_index[0].astype(jnp.int32)
    src = edge_index[1].astype(jnp.int32)
    w = edge_weight.astype(jnp.float32)
    pad = e_pad - e
    src_p = jnp.pad(src, (0, pad))
    dst_p = jnp.pad(dst, (0, pad))
    w_p = jnp.pad(w, (0, pad))
    src2a = jnp.stack([src_p, src_p + n]).reshape(2, NS, ct, CHUNK)
    src2b = jnp.stack([src_p, src_p + n_pad]).reshape(2, NS, ct, CHUNK)
    dst_r = dst_p.reshape(NS, ct, CHUNK)
    w_r = w_p.reshape(NS, ct * CHUNK)
    zrows = jnp.zeros((n_pad // NS, HALF), jnp.float32)

    x1 = _mm_stacked(feat, W1, relu=False)
    y1 = _spmm_sc(n_pad, x1.reshape(2 * n, HALF), src2a, dst_r, w_r, zrows)
    x2 = _mm_stacked(y1, W2, relu=True)
    y2 = _spmm_sc(n_pad, x2.reshape(2 * n_pad, HALF), src2b, dst_r, w_r, zrows)
    return y2[:n]
